# baseline (speedup 1.0000x reference)
import functools

import jax
import jax.numpy as jnp
from jax import lax
from jax.experimental import pallas as pl
from jax.experimental.pallas import tpu as pltpu
from jax.experimental.pallas import tpu_sc as plsc

NUM_CLASSES = 210
SEM_DIM = 512
FC_IN = 2048
BATCH = 16384
K = 10


BM = 1024


def _tc_body(avg_ref, w_ref, b_ref, tags_ref, idx_ref):
    avg = avg_ref[...]
    w = w_ref[...]
    tags = lax.dot_general(
        w, avg,
        dimension_numbers=(((1,), (1,)), ((), ())),
        preferred_element_type=jnp.float32,
    ) + b_ref[...]
    tags_ref[...] = tags

    iota = lax.broadcasted_iota(jnp.int32, (NUM_CLASSES, BM), 0)
    work = tags
    rows = []
    for _ in range(K):
        m = jnp.max(work, axis=0, keepdims=True)
        cand = jnp.where(work == m, iota, NUM_CLASSES)
        a = jnp.min(cand, axis=0, keepdims=True)
        rows.append(a)
        work = jnp.where(iota == a, -jnp.inf, work)
    idx_ref[...] = jnp.concatenate(rows, axis=0)


def _tc_call(avg_features, W, b):
    grid = BATCH // BM
    return pl.pallas_call(
        _tc_body,
        grid=(grid,),
        in_specs=[
            pl.BlockSpec((BM, FC_IN), lambda i: (i, 0)),
            pl.BlockSpec((NUM_CLASSES, FC_IN), lambda i: (0, 0)),
            pl.BlockSpec((NUM_CLASSES, 1), lambda i: (0, 0)),
        ],
        out_specs=[
            pl.BlockSpec((NUM_CLASSES, BM), lambda i: (0, i)),
            pl.BlockSpec((K, BM), lambda i: (0, i)),
        ],
        out_shape=[
            jax.ShapeDtypeStruct((NUM_CLASSES, BATCH), jnp.float32),
            jax.ShapeDtypeStruct((K, BATCH), jnp.int32),
        ],
    )(avg_features, W, b.reshape(NUM_CLASSES, 1))


_NBR = 128
_SB = 64
_SCH = _SB * 4


def _sc_gather_kernel(table_hbm, idx_hbm, out_hbm,
                      idx_v, idx4a, idx4b, rva, rvb, sem_g, sem_o,
                      *, rows_per_worker, num_cores):
    wid = lax.axis_index("s") * num_cores + lax.axis_index("c")
    b_base = wid * rows_per_worker
    nchunks = rows_per_worker // _NBR
    lane = lax.iota(jnp.int32, 16)
    bufs = ((idx4a, rva), (idx4b, rvb))

    def body(i, carry):
        b0 = b_base + i * _NBR
        pltpu.sync_copy(idx_hbm.at[:, pl.ds(b0, _NBR)], idx_v)
        for r in range(2 * K):
            k, h = r >> 1, r & 1
            idx4_v, rv = bufs[r & 1]
            dst = out_hbm.at[k, pl.ds(b0 + h * _SB, _SB), :]
            if r >= 2:
                pltpu.make_async_copy(rv.reshape(_SB, SEM_DIM), dst,
                                      sem_o).wait()
            else:
                @pl.when(i > 0)
                def _():
                    pltpu.make_async_copy(rv.reshape(_SB, SEM_DIM), dst,
                                          sem_o).wait()
            for g in range(_SCH // 16):
                bl = lax.shift_right_logical(lane, 2) + (g * 4 + h * _SB)
                kv = lax.bitwise_and(lane, 0) + k
                src = plsc.load_gather(idx_v, [kv, bl])
                idx4_v[pl.ds(g * 16, 16)] = (
                    lax.shift_left(src, 2) + lax.bitwise_and(lane, 3))
            cps = [pltpu.async_copy(
                table_hbm.at[idx4_v.at[pl.ds(p, 128)]],
                rv.at[pl.ds(p, 128)], sem_g) for p in range(0, _SCH, 128)]
            for cp in cps:
                cp.wait()
            pltpu.async_copy(rv.reshape(_SB, SEM_DIM), dst, sem_o)
        return carry

    lax.fori_loop(0, nchunks, body, 0)
    for r in range(2):
        _, rv = bufs[r]
        pltpu.make_async_copy(
            rv.reshape(_SB, SEM_DIM),
            out_hbm.at[0, pl.ds(b_base, _SB), :], sem_o).wait()


def _sc_gather(table840, idx_t):
    info = plsc.get_sparse_core_info()
    nw = info.num_cores * info.num_subcores
    rows_per_worker = BATCH // nw
    mesh = plsc.VectorSubcoreMesh(core_axis_name="c", subcore_axis_name="s")
    kern = pl.kernel(
        functools.partial(_sc_gather_kernel,
                          rows_per_worker=rows_per_worker,
                          num_cores=info.num_cores),
        out_type=jax.ShapeDtypeStruct((K, BATCH, SEM_DIM), jnp.float32),
        mesh=mesh,
        scratch_types=[
            pltpu.VMEM((K, _NBR), jnp.int32),
            pltpu.VMEM((_SCH,), jnp.int32),
            pltpu.VMEM((_SCH,), jnp.int32),
            pltpu.VMEM((_SCH, 128), jnp.float32),
            pltpu.VMEM((_SCH, 128), jnp.float32),
            pltpu.SemaphoreType.DMA,
            pltpu.SemaphoreType.DMA,
        ],
        compiler_params=pltpu.CompilerParams(use_tc_tiling_on_sc=True,
                                             needs_layout_passes=False),
    )
    return kern(table840, idx_t)


def kernel(avg_features, W, b, embed_table):
    tags_t, idx_t = _tc_call(avg_features, W, b)
    table840 = embed_table.reshape(NUM_CLASSES * 4, SEM_DIM // 4)
    semantic = _sc_gather(table840, idx_t)
    return tags_t.T, semantic.transpose(1, 0, 2)


# speedup vs baseline: 1.7960x; 1.7960x over previous
import functools

import jax
import jax.numpy as jnp
from jax import lax
from jax.experimental import pallas as pl
from jax.experimental.pallas import tpu as pltpu
from jax.experimental.pallas import tpu_sc as plsc

NUM_CLASSES = 210
SEM_DIM = 512
FC_IN = 2048
BATCH = 16384
K = 10


BM = 1024


def _tc_body(avg_ref, w_ref, b_ref, tags_ref, idx_ref):
    avg = avg_ref[...]
    w = w_ref[...]
    tags = lax.dot_general(
        w, avg,
        dimension_numbers=(((1,), (1,)), ((), ())),
        preferred_element_type=jnp.float32,
    ) + b_ref[...]
    tags_ref[...] = tags

    iota = lax.broadcasted_iota(jnp.int32, (NUM_CLASSES, BM), 0)
    work = tags
    rows = []
    for _ in range(K):
        m = jnp.max(work, axis=0, keepdims=True)
        cand = jnp.where(work == m, iota, NUM_CLASSES)
        a = jnp.min(cand, axis=0, keepdims=True)
        rows.append(a)
        work = jnp.where(iota == a, -jnp.inf, work)
    idx_ref[...] = jnp.concatenate(rows, axis=0)


def _tc_call(avg_features, W, b):
    grid = BATCH // BM
    return pl.pallas_call(
        _tc_body,
        grid=(grid,),
        in_specs=[
            pl.BlockSpec((BM, FC_IN), lambda i: (i, 0)),
            pl.BlockSpec((NUM_CLASSES, FC_IN), lambda i: (0, 0)),
            pl.BlockSpec((NUM_CLASSES, 1), lambda i: (0, 0)),
        ],
        out_specs=[
            pl.BlockSpec((NUM_CLASSES, BM), lambda i: (0, i)),
            pl.BlockSpec((K, BM), lambda i: (0, i)),
        ],
        out_shape=[
            jax.ShapeDtypeStruct((NUM_CLASSES, BATCH), jnp.float32),
            jax.ShapeDtypeStruct((K, BATCH), jnp.int32),
        ],
    )(avg_features, W, b.reshape(NUM_CLASSES, 1))


_NBR = 128
_SB = 64
_SCH = _SB * 4


def _sc_gather_kernel(table_hbm, idx_hbm, out_hbm,
                      idx_v, idx4a, idx4b, rva, rvb, table_sp, sem_g, sem_o,
                      *, rows_per_worker, num_cores):
    wid = lax.axis_index("s") * num_cores + lax.axis_index("c")
    b_base = wid * rows_per_worker
    nchunks = rows_per_worker // _NBR
    lane = lax.iota(jnp.int32, 16)
    bufs = ((idx4a, rva), (idx4b, rvb))

    @pl.when(lax.axis_index("s") == 0)
    def _():
        pltpu.sync_copy(table_hbm, table_sp)
    plsc.subcore_barrier()

    def body(i, carry):
        b0 = b_base + i * _NBR
        pltpu.sync_copy(idx_hbm.at[:, pl.ds(b0, _NBR)], idx_v)
        for r in range(2 * K):
            k, h = r >> 1, r & 1
            idx4_v, rv = bufs[r & 1]
            dst = out_hbm.at[k, pl.ds(b0 + h * _SB, _SB), :]
            if r >= 2:
                pltpu.make_async_copy(rv.reshape(_SB, SEM_DIM), dst,
                                      sem_o).wait()
            else:
                @pl.when(i > 0)
                def _():
                    pltpu.make_async_copy(rv.reshape(_SB, SEM_DIM), dst,
                                          sem_o).wait()
            for g in range(_SCH // 16):
                bl = lax.shift_right_logical(lane, 2) + (g * 4 + h * _SB)
                kv = lax.bitwise_and(lane, 0) + k
                src = plsc.load_gather(idx_v, [kv, bl])
                idx4_v[pl.ds(g * 16, 16)] = (
                    lax.shift_left(src, 2) + lax.bitwise_and(lane, 3))
            cps = [pltpu.async_copy(
                table_sp.at[idx4_v.at[pl.ds(p, 128)]],
                rv.at[pl.ds(p, 128)], sem_g) for p in range(0, _SCH, 128)]
            for cp in cps:
                cp.wait()
            pltpu.async_copy(rv.reshape(_SB, SEM_DIM), dst, sem_o)
        return carry

    lax.fori_loop(0, nchunks, body, 0)
    for r in range(2):
        _, rv = bufs[r]
        pltpu.make_async_copy(
            rv.reshape(_SB, SEM_DIM),
            out_hbm.at[0, pl.ds(b_base, _SB), :], sem_o).wait()


def _sc_gather(table840, idx_t):
    info = plsc.get_sparse_core_info()
    nw = info.num_cores * info.num_subcores
    rows_per_worker = BATCH // nw
    mesh = plsc.VectorSubcoreMesh(core_axis_name="c", subcore_axis_name="s")
    kern = pl.kernel(
        functools.partial(_sc_gather_kernel,
                          rows_per_worker=rows_per_worker,
                          num_cores=info.num_cores),
        out_type=jax.ShapeDtypeStruct((K, BATCH, SEM_DIM), jnp.float32),
        mesh=mesh,
        scratch_types=[
            pltpu.VMEM((K, _NBR), jnp.int32),
            pltpu.VMEM((_SCH,), jnp.int32),
            pltpu.VMEM((_SCH,), jnp.int32),
            pltpu.VMEM((_SCH, 128), jnp.float32),
            pltpu.VMEM((_SCH, 128), jnp.float32),
            pltpu.VMEM_SHARED((NUM_CLASSES * 4, SEM_DIM // 4), jnp.float32),
            pltpu.SemaphoreType.DMA,
            pltpu.SemaphoreType.DMA,
        ],
        compiler_params=pltpu.CompilerParams(use_tc_tiling_on_sc=True,
                                             needs_layout_passes=False),
    )
    return kern(table840, idx_t)


def kernel(avg_features, W, b, embed_table):
    tags_t, idx_t = _tc_call(avg_features, W, b)
    table840 = embed_table.reshape(NUM_CLASSES * 4, SEM_DIM // 4)
    semantic = _sc_gather(table840, idx_t)
    return tags_t.T, semantic.transpose(1, 0, 2)


# speedup vs baseline: 1.8120x; 1.0089x over previous
import functools

import jax
import jax.numpy as jnp
from jax import lax
from jax.experimental import pallas as pl
from jax.experimental.pallas import tpu as pltpu
from jax.experimental.pallas import tpu_sc as plsc

NUM_CLASSES = 210
SEM_DIM = 512
FC_IN = 2048
BATCH = 16384
K = 10


BM = 1024


def _tc_body(avg_ref, w_ref, b_ref, tags_ref, idx_ref):
    avg = avg_ref[...]
    w = w_ref[...]
    tags = lax.dot_general(
        w, avg,
        dimension_numbers=(((1,), (1,)), ((), ())),
        preferred_element_type=jnp.float32,
    ) + b_ref[...]
    tags_ref[...] = tags

    iota = lax.broadcasted_iota(
        jnp.int32, (NUM_CLASSES, BM), 0).astype(jnp.float32)
    work = tags
    rows = []
    for _ in range(K):
        m = jnp.max(work, axis=0, keepdims=True)
        cand = jnp.where(work == m, iota, float(NUM_CLASSES))
        a = jnp.min(cand, axis=0, keepdims=True)
        rows.append(a)
        work = jnp.where(iota == a, -jnp.inf, work)
    idx_ref[...] = jnp.concatenate(rows, axis=0).astype(jnp.int32)


def _tc_call(avg_features, W, b):
    grid = BATCH // BM
    return pl.pallas_call(
        _tc_body,
        grid=(grid,),
        in_specs=[
            pl.BlockSpec((BM, FC_IN), lambda i: (i, 0)),
            pl.BlockSpec((NUM_CLASSES, FC_IN), lambda i: (0, 0)),
            pl.BlockSpec((NUM_CLASSES, 1), lambda i: (0, 0)),
        ],
        out_specs=[
            pl.BlockSpec((NUM_CLASSES, BM), lambda i: (0, i)),
            pl.BlockSpec((K, BM), lambda i: (0, i)),
        ],
        out_shape=[
            jax.ShapeDtypeStruct((NUM_CLASSES, BATCH), jnp.float32),
            jax.ShapeDtypeStruct((K, BATCH), jnp.int32),
        ],
    )(avg_features, W, b.reshape(NUM_CLASSES, 1))


_NBR = 128
_SB = 64
_SCH = _SB * 4


def _sc_gather_kernel(table_hbm, idx_hbm, out_hbm,
                      idx_v, idx4a, idx4b, rva, rvb, table_sp, sem_g, sem_o,
                      *, rows_per_worker, num_cores):
    wid = lax.axis_index("s") * num_cores + lax.axis_index("c")
    b_base = wid * rows_per_worker
    nchunks = rows_per_worker // _NBR
    lane = lax.iota(jnp.int32, 16)
    bufs = ((idx4a, rva), (idx4b, rvb))

    @pl.when(lax.axis_index("s") == 0)
    def _():
        pltpu.sync_copy(table_hbm, table_sp)
    plsc.subcore_barrier()

    def body(i, carry):
        b0 = b_base + i * _NBR
        pltpu.sync_copy(idx_hbm.at[:, pl.ds(b0, _NBR)], idx_v)
        for r in range(2 * K):
            k, h = r >> 1, r & 1
            idx4_v, rv = bufs[r & 1]
            dst = out_hbm.at[k, pl.ds(b0 + h * _SB, _SB), :]
            if r >= 2:
                pltpu.make_async_copy(rv.reshape(_SB, SEM_DIM), dst,
                                      sem_o).wait()
            else:
                @pl.when(i > 0)
                def _():
                    pltpu.make_async_copy(rv.reshape(_SB, SEM_DIM), dst,
                                          sem_o).wait()
            for g in range(_SCH // 16):
                bl = lax.shift_right_logical(lane, 2) + (g * 4 + h * _SB)
                kv = lax.bitwise_and(lane, 0) + k
                src = plsc.load_gather(idx_v, [kv, bl])
                idx4_v[pl.ds(g * 16, 16)] = (
                    lax.shift_left(src, 2) + lax.bitwise_and(lane, 3))
            cps = [pltpu.async_copy(
                table_sp.at[idx4_v.at[pl.ds(p, 128)]],
                rv.at[pl.ds(p, 128)], sem_g) for p in range(0, _SCH, 128)]
            for cp in cps:
                cp.wait()
            pltpu.async_copy(rv.reshape(_SB, SEM_DIM), dst, sem_o)
        return carry

    lax.fori_loop(0, nchunks, body, 0)
    for r in range(2):
        _, rv = bufs[r]
        pltpu.make_async_copy(
            rv.reshape(_SB, SEM_DIM),
            out_hbm.at[0, pl.ds(b_base, _SB), :], sem_o).wait()


def _sc_gather(table840, idx_t):
    info = plsc.get_sparse_core_info()
    nw = info.num_cores * info.num_subcores
    rows_per_worker = BATCH // nw
    mesh = plsc.VectorSubcoreMesh(core_axis_name="c", subcore_axis_name="s")
    kern = pl.kernel(
        functools.partial(_sc_gather_kernel,
                          rows_per_worker=rows_per_worker,
                          num_cores=info.num_cores),
        out_type=jax.ShapeDtypeStruct((K, BATCH, SEM_DIM), jnp.float32),
        mesh=mesh,
        scratch_types=[
            pltpu.VMEM((K, _NBR), jnp.int32),
            pltpu.VMEM((_SCH,), jnp.int32),
            pltpu.VMEM((_SCH,), jnp.int32),
            pltpu.VMEM((_SCH, 128), jnp.float32),
            pltpu.VMEM((_SCH, 128), jnp.float32),
            pltpu.VMEM_SHARED((NUM_CLASSES * 4, SEM_DIM // 4), jnp.float32),
            pltpu.SemaphoreType.DMA,
            pltpu.SemaphoreType.DMA,
        ],
        compiler_params=pltpu.CompilerParams(use_tc_tiling_on_sc=True,
                                             needs_layout_passes=False),
    )
    return kern(table840, idx_t)


def kernel(avg_features, W, b, embed_table):
    tags_t, idx_t = _tc_call(avg_features, W, b)
    table840 = embed_table.reshape(NUM_CLASSES * 4, SEM_DIM // 4)
    semantic = _sc_gather(table840, idx_t)
    return tags_t.T, semantic.transpose(1, 0, 2)
